# baseline (speedup 1.0000x reference)
import functools

import jax
import jax.numpy as jnp
from jax import lax
from jax.experimental import pallas as pl
from jax.experimental.pallas import tpu as pltpu
from jax.experimental.pallas import tpu_sc as plsc

_NC = 2
_NS = 16
_LANES = 16
_CHUNK = 32
_NB_G = 2
_NB_X = 3
_STEP = 6
_GRP = 8


@functools.partial(jax.jit, static_argnames=("n_rows", "d_half"))
def _sc_lookup_add(x2d, c1, c2, table1, table2, n_rows, d_half):
    d_model = 2 * d_half
    n_workers = _NC * _NS
    rows_per_worker = n_rows // n_workers
    n_chunks = rows_per_worker // _CHUNK
    assert n_rows % (n_workers * _CHUNK) == 0
    n_slots = -(-n_chunks // _STEP) * _STEP

    mesh = plsc.VectorSubcoreMesh(core_axis_name="c", subcore_axis_name="s")

    @functools.partial(
        pl.kernel,
        mesh=mesh,
        out_type=jax.ShapeDtypeStruct((n_rows, d_model), jnp.float32),
        scratch_types=[
            pltpu.VMEM((rows_per_worker,), jnp.int32),
            pltpu.VMEM((rows_per_worker,), jnp.int32),
            pltpu.VMEM((_NB_G, _CHUNK, d_half), jnp.float32),
            pltpu.VMEM((_NB_G, _CHUNK, d_half), jnp.float32),
            pltpu.VMEM((_NB_X, _CHUNK, d_model), jnp.float32),
            pltpu.SemaphoreType.DMA((_NB_G,)),
            pltpu.SemaphoreType.DMA((_NB_X,)),
            pltpu.SemaphoreType.DMA((_NB_X,)),
        ],
    )
    def k(x_hbm, c1_hbm, c2_hbm, t1_hbm, t2_hbm, out_hbm,
          idx1_v, idx2_v, buf1, buf2, xbuf, sem_g, sem_x, sem_st):
        wid = lax.axis_index("s") * _NC + lax.axis_index("c")
        base = wid * rows_per_worker
        pltpu.sync_copy(c1_hbm.at[pl.ds(base, rows_per_worker)], idx1_v)
        pltpu.sync_copy(c2_hbm.at[pl.ds(base, rows_per_worker)], idx2_v)

        def start_gathers(c, bg):
            idx = pl.ds(c * _CHUNK, _CHUNK)
            pltpu.async_copy(t1_hbm.at[idx1_v.at[idx]], buf1.at[bg],
                             sem_g.at[bg])
            pltpu.async_copy(t2_hbm.at[idx2_v.at[idx]], buf2.at[bg],
                             sem_g.at[bg])

        def wait_gathers(bg):
            idx = pl.ds(0, _CHUNK)
            pltpu.make_async_copy(t1_hbm.at[idx1_v.at[idx]], buf1.at[bg],
                                  sem_g.at[bg]).wait()
            pltpu.make_async_copy(t2_hbm.at[idx2_v.at[idx]], buf2.at[bg],
                                  sem_g.at[bg]).wait()

        def start_xload(c, bx):
            pltpu.async_copy(x_hbm.at[pl.ds(base + c * _CHUNK, _CHUNK)],
                             xbuf.at[bx], sem_x.at[bx])

        def wait_xload(bx):
            pltpu.make_async_copy(x_hbm.at[pl.ds(base, _CHUNK)],
                                  xbuf.at[bx], sem_x.at[bx]).wait()

        def start_store(c, bx):
            pltpu.async_copy(xbuf.at[bx],
                             out_hbm.at[pl.ds(base + c * _CHUNK, _CHUNK)],
                             sem_st.at[bx])

        def wait_store(bx):
            pltpu.make_async_copy(xbuf.at[bx],
                                  out_hbm.at[pl.ds(base, _CHUNK)],
                                  sem_st.at[bx]).wait()

        start_gathers(0, 0)
        start_xload(0, 0)

        n_sl = d_half // _LANES

        def group_body(p, carry):
            for kslot in range(_STEP):
                bg = kslot % _NB_G
                bx = kslot % _NB_X
                bg1 = (kslot + 1) % _NB_G
                bx1 = (kslot + 1) % _NB_X
                c = p * _STEP + kslot

                @pl.when(c + 1 < n_chunks)
                def _():
                    start_gathers(c + 1, bg1)

                    @pl.when(c + 1 >= _NB_X)
                    def _():
                        wait_store(bx1)

                    start_xload(c + 1, bx1)

                @pl.when(c < n_chunks)
                def _():
                    wait_gathers(bg)
                    wait_xload(bx)

                    @plsc.parallel_loop(0, _CHUNK, step=1)
                    def row_body(r):
                        for g in range(0, n_sl, _GRP):
                            v1s = [buf1[bg, r, pl.ds((g + j) * _LANES, _LANES)]
                                   for j in range(_GRP)]
                            v2s = [buf2[bg, r, pl.ds((g + j) * _LANES, _LANES)]
                                   for j in range(_GRP)]
                            for j in range(_GRP):
                                plsc.addupdate(
                                    xbuf.at[bx, r,
                                            pl.ds((g + j) * _LANES, _LANES)],
                                    v1s[j])
                            for j in range(_GRP):
                                plsc.addupdate(
                                    xbuf.at[bx, r,
                                            pl.ds(d_half + (g + j) * _LANES,
                                                  _LANES)],
                                    v2s[j])

                    start_store(c, bx)
            return carry

        lax.fori_loop(0, n_slots // _STEP, group_body, 0)
        for i in range(_NB_X):
            wait_store((n_chunks - 1 - i) % _NB_X)

    return k(x2d, c1, c2, table1, table2)


def kernel(x, coord1, coord2, table1, table2):
    b, s, d_model = x.shape
    n_rows = b * s
    d_half = table1.shape[1]
    x2d = x.reshape(n_rows, d_model)
    c1 = coord1.reshape(n_rows).astype(jnp.int32)
    c2 = coord2.reshape(n_rows).astype(jnp.int32)
    out = _sc_lookup_add(x2d, c1, c2, table1, table2, n_rows, d_half)
    return out.reshape(b, s, d_model)


# speedup vs baseline: 1.0374x; 1.0374x over previous
import functools

import jax
import jax.numpy as jnp
from jax import lax
from jax.experimental import pallas as pl
from jax.experimental.pallas import tpu as pltpu
from jax.experimental.pallas import tpu_sc as plsc

_NC = 2
_NS = 16
_LANES = 16
_CHUNK = 16
_NBUF = 4
_PREFETCH = 2
_GRP = 8


@functools.partial(jax.jit, static_argnames=("n_rows", "d_half"))
def _sc_lookup_add(x2d, c1, c2, table1, table2, n_rows, d_half):
    d_model = 2 * d_half
    n_workers = _NC * _NS
    rows_per_worker = n_rows // n_workers
    n_chunks = rows_per_worker // _CHUNK
    assert n_rows % (n_workers * _CHUNK) == 0
    assert n_chunks % _NBUF == 0

    mesh = plsc.VectorSubcoreMesh(core_axis_name="c", subcore_axis_name="s")

    @functools.partial(
        pl.kernel,
        mesh=mesh,
        out_type=jax.ShapeDtypeStruct((n_rows, d_model), jnp.float32),
        scratch_types=[
            pltpu.VMEM((rows_per_worker,), jnp.int32),
            pltpu.VMEM((rows_per_worker,), jnp.int32),
            pltpu.VMEM((_NBUF, _CHUNK, d_half), jnp.float32),
            pltpu.VMEM((_NBUF, _CHUNK, d_half), jnp.float32),
            pltpu.VMEM((_NBUF, _CHUNK, d_model), jnp.float32),
            pltpu.SemaphoreType.DMA((_NBUF,)),
            pltpu.SemaphoreType.DMA((_NBUF,)),
        ],
    )
    def k(x_hbm, c1_hbm, c2_hbm, t1_hbm, t2_hbm, out_hbm,
          idx1_v, idx2_v, buf1, buf2, xbuf, sem_ld, sem_st):
        wid = lax.axis_index("s") * _NC + lax.axis_index("c")
        base = wid * rows_per_worker
        pltpu.sync_copy(c1_hbm.at[pl.ds(base, rows_per_worker)], idx1_v)
        pltpu.sync_copy(c2_hbm.at[pl.ds(base, rows_per_worker)], idx2_v)

        def start_gathers(c, b):
            pltpu.async_copy(
                t1_hbm.at[idx1_v.at[pl.ds(c * _CHUNK, _CHUNK)]],
                buf1.at[b], sem_ld.at[b])
            pltpu.async_copy(
                t2_hbm.at[idx2_v.at[pl.ds(c * _CHUNK, _CHUNK)]],
                buf2.at[b], sem_ld.at[b])

        def start_xload(c, b):
            pltpu.async_copy(
                x_hbm.at[pl.ds(base + c * _CHUNK, _CHUNK)],
                xbuf.at[b], sem_ld.at[b])

        def wait_loads(b):
            pltpu.make_async_copy(
                t1_hbm.at[idx1_v.at[pl.ds(0, _CHUNK)]],
                buf1.at[b], sem_ld.at[b]).wait()
            pltpu.make_async_copy(
                t2_hbm.at[idx2_v.at[pl.ds(0, _CHUNK)]],
                buf2.at[b], sem_ld.at[b]).wait()
            pltpu.make_async_copy(
                x_hbm.at[pl.ds(base, _CHUNK)],
                xbuf.at[b], sem_ld.at[b]).wait()

        def start_store(c, b):
            pltpu.async_copy(
                xbuf.at[b], out_hbm.at[pl.ds(base + c * _CHUNK, _CHUNK)],
                sem_st.at[b])

        def wait_store(b):
            pltpu.make_async_copy(
                xbuf.at[b], out_hbm.at[pl.ds(base, _CHUNK)],
                sem_st.at[b]).wait()

        for c0 in range(_PREFETCH):
            start_gathers(c0, c0 % _NBUF)
            start_xload(c0, c0 % _NBUF)

        def group_body(p, carry):
            for b in range(_NBUF):
                c = p * _NBUF + b
                cpf = c + _PREFETCH
                bpf = (b + _PREFETCH) % _NBUF

                @pl.when(cpf < n_chunks)
                def _():
                    @pl.when(cpf >= _NBUF)
                    def _():
                        wait_store(bpf)

                    start_gathers(cpf, bpf)
                    start_xload(cpf, bpf)

                wait_loads(b)

                n_sl = d_half // _LANES

                @plsc.parallel_loop(0, _CHUNK, step=1)
                def row_body(r):
                    for g in range(0, n_sl, _GRP):
                        v1s = [buf1[b, r, pl.ds((g + k) * _LANES, _LANES)]
                               for k in range(_GRP)]
                        v2s = [buf2[b, r, pl.ds((g + k) * _LANES, _LANES)]
                               for k in range(_GRP)]
                        for k in range(_GRP):
                            plsc.addupdate(
                                xbuf.at[b, r, pl.ds((g + k) * _LANES, _LANES)],
                                v1s[k])
                        for k in range(_GRP):
                            plsc.addupdate(
                                xbuf.at[b, r,
                                        pl.ds(d_half + (g + k) * _LANES, _LANES)],
                                v2s[k])
                start_store(c, b)
            return carry

        lax.fori_loop(0, n_chunks // _NBUF, group_body, 0)
        for b in range(_NBUF):
            wait_store(b)

    return k(x2d, c1, c2, table1, table2)


def kernel(x, coord1, coord2, table1, table2):
    b, s, d_model = x.shape
    n_rows = b * s
    d_half = table1.shape[1]
    x2d = x.reshape(n_rows, d_model)
    c1 = coord1.reshape(n_rows).astype(jnp.int32)
    c2 = coord2.reshape(n_rows).astype(jnp.int32)
    out = _sc_lookup_add(x2d, c1, c2, table1, table2, n_rows, d_half)
    return out.reshape(b, s, d_model)
